# baseline (speedup 1.0000x reference)
import functools

import jax
import jax.numpy as jnp
from jax import lax
from jax.experimental import pallas as pl
from jax.experimental.pallas import tpu as pltpu
from jax.experimental.pallas import tpu_sc as plsc

N, E, D = 10000, 320000, 128
DH = D // 2
NC, NS = 2, 16
K = 80
NB = 5
CPT = E // (K * NS)
APAD = 10112
RB = 400


@functools.lru_cache(maxsize=None)
def _edge_pass(table_rows, off):
    mesh = plsc.VectorSubcoreMesh(core_axis_name="c", subcore_axis_name="s",
                                  num_cores=NC, num_subcores=NS)
    out_type = [jax.ShapeDtypeStruct((NC, N, DH), jnp.float32)] * 2
    scratch = (
        [pltpu.VMEM((CPT, K), jnp.int32)] * 3
        + [pltpu.VMEM((K, DH), jnp.float32)] * NB
        + [pltpu.VMEM_SHARED((APAD, DH), jnp.float32)]
        + [pltpu.SemaphoreType.DMA] * (NB + 1)
    )

    @functools.partial(
        pl.kernel, mesh=mesh, out_type=out_type, scratch_types=scratch,
        compiler_params=pltpu.CompilerParams(use_tc_tiling_on_sc=False))
    def run(*refs):
        (tbl_l, tbl_r, src_h, dst_h, km_h, accl_o, accr_o,
         srcv, dstv, kmv, *rest) = refs
        rows = rest[:NB]
        acc_sh = rest[NB]
        sem_g = rest[NB + 1:2 * NB + 1]
        sem_s = rest[2 * NB + 1]
        cid = lax.axis_index("c")
        sid = lax.axis_index("s")

        pltpu.sync_copy(src_h.at[sid], srcv)
        pltpu.sync_copy(dst_h.at[sid], dstv)
        pltpu.sync_copy(km_h.at[sid], kmv)

        gsel = K // 16
        goff = jnp.full((16,), cid * off, dtype=jnp.int32)

        lanes = lax.iota(jnp.int32, 16)

        def ibody(i, _):
            c = i // gsel
            j = (i % gsel) * 16
            if off:
                srcv[c, pl.ds(j, 16)] = srcv[c, pl.ds(j, 16)] + goff
            keep = jnp.maximum(kmv[c, pl.ds(j, 16)], 1 - cid)
            dummy = (N + (i % 7) * 16) + lanes
            dstv[c, pl.ds(j, 16)] = (keep * dstv[c, pl.ds(j, 16)]
                                     + (1 - keep) * dummy)
            return 0

        lax.fori_loop(0, CPT * gsel, ibody, 0)

        z0 = sid * (APAD // NS)
        stripe = APAD // NS

        def one_phase(table, acc_o):
            def zbody(i, _):
                rows[0][i // (DH // 16),
                        pl.ds((i % (DH // 16)) * 16, 16)] = (
                    jnp.zeros((16,), jnp.float32))
                return 0

            lax.fori_loop(0, K * (DH // 16), zbody, 0)
            for t in range(stripe // K):
                pltpu.sync_copy(rows[0], acc_sh.at[pl.ds(z0 + t * K, K)])
            if stripe % K:
                pltpu.sync_copy(
                    rows[0].at[pl.ds(0, stripe % K)],
                    acc_sh.at[pl.ds(z0 + (stripe // K) * K, stripe % K)])
            plsc.subcore_barrier()

            for b in range(NB):
                pltpu.async_copy(table.at[srcv.at[b]], rows[b], sem_g[b])

            def mbody(p, _):
                for b in range(NB):
                    c = p * NB + b
                    pltpu.make_async_copy(table.at[srcv.at[c]], rows[b],
                                          sem_g[b]).wait()
                    pltpu.async_copy(rows[b], acc_sh.at[dstv.at[c]],
                                     sem_s, add=True).wait()
                    cn = jnp.minimum(c + NB, CPT - 1)
                    pltpu.async_copy(table.at[srcv.at[cn]], rows[b],
                                     sem_g[b])
                return 0

            lax.fori_loop(0, CPT // NB, mbody, 0)
            for b in range(NB):
                pltpu.make_async_copy(table.at[srcv.at[CPT - 1]], rows[b],
                                      sem_g[b]).wait()
            plsc.subcore_barrier()

            @pl.when(sid < NS - 1)
            def _():
                w0 = sid * 640
                pltpu.sync_copy(acc_sh.at[pl.ds(w0, 640)],
                                acc_o.at[cid, pl.ds(w0, 640)])

            @pl.when(sid == NS - 1)
            def _():
                pltpu.sync_copy(acc_sh.at[pl.ds(9600, N - 9600)],
                                acc_o.at[cid, pl.ds(9600, N - 9600)])

        one_phase(tbl_l, accl_o)
        one_phase(tbl_r, accr_o)

    return run


@functools.lru_cache(maxsize=None)
def _deg_pass():
    mesh = plsc.VectorSubcoreMesh(core_axis_name="c", subcore_axis_name="s",
                                  num_cores=NC, num_subcores=NS)
    scratch = (
        [pltpu.VMEM((CPT, K), jnp.int32)] * 2
        + [pltpu.VMEM((K, 16), jnp.float32)] * 2
        + [pltpu.VMEM_SHARED((APAD, 16), jnp.float32)]
        + [pltpu.SemaphoreType.DMA] * NB
    )

    @functools.partial(
        pl.kernel, mesh=mesh,
        out_type=[jax.ShapeDtypeStruct((NC, N, 16), jnp.float32)],
        scratch_types=scratch,
        compiler_params=pltpu.CompilerParams(use_tc_tiling_on_sc=False))
    def run(dst_h, km_h, deg_o, dstv, kmv, onesv, zdeg, deg_sh, *sem_d):
        cid = lax.axis_index("c")
        sid = lax.axis_index("s")
        pltpu.sync_copy(dst_h.at[sid], dstv)
        pltpu.sync_copy(km_h.at[sid], kmv)
        gsel = K // 16

        lanes = lax.iota(jnp.int32, 16)

        def ibody(i, _):
            c = i // gsel
            j = (i % gsel) * 16
            keep = jnp.maximum(kmv[c, pl.ds(j, 16)], 1 - cid)
            dummy = (N + (i % 7) * 16) + lanes
            dstv[c, pl.ds(j, 16)] = (keep * dstv[c, pl.ds(j, 16)]
                                     + (1 - keep) * dummy)
            return 0

        lax.fori_loop(0, CPT * gsel, ibody, 0)
        onerow = jnp.maximum(1 - lax.iota(jnp.int32, 16), 0).astype(
            jnp.float32)

        def obody(i, _):
            onesv[i, pl.ds(0, 16)] = onerow
            zdeg[i, pl.ds(0, 16)] = jnp.zeros((16,), jnp.float32)
            return 0

        lax.fori_loop(0, K, obody, 0)
        z0 = sid * (APAD // NS)
        stripe = APAD // NS
        for t in range(stripe // K):
            pltpu.sync_copy(zdeg, deg_sh.at[pl.ds(z0 + t * K, K)])
        if stripe % K:
            pltpu.sync_copy(
                zdeg.at[pl.ds(0, stripe % K)],
                deg_sh.at[pl.ds(z0 + (stripe // K) * K, stripe % K)])
        plsc.subcore_barrier()

        for b in range(NB):
            pltpu.async_copy(onesv, deg_sh.at[dstv.at[b]], sem_d[b],
                             add=True)

        def dbody(p, _):
            for b in range(NB):
                c = p * NB + b
                pltpu.make_async_copy(onesv, deg_sh.at[dstv.at[c]],
                                      sem_d[b]).wait()

                @pl.when(c + NB < CPT)
                def _():
                    pltpu.async_copy(onesv, deg_sh.at[dstv.at[c + NB]],
                                     sem_d[b], add=True)

            return 0

        lax.fori_loop(0, CPT // NB, dbody, 0)
        plsc.subcore_barrier()

        @pl.when(sid < NS - 1)
        def _():
            w0 = sid * 640
            pltpu.sync_copy(deg_sh.at[pl.ds(w0, 640)],
                            deg_o.at[cid, pl.ds(w0, 640)])

        @pl.when(sid == NS - 1)
        def _():
            pltpu.sync_copy(deg_sh.at[pl.ds(9600, N - 9600)],
                            deg_o.at[cid, pl.ds(9600, N - 9600)])

    return run


def _encode_body(x_r, accl_r, accr_r, deg_r, nt_r, w_r, b_r, a_r,
                 hcl_r, hcr_r, ph_r):
    xv = x_r[...]
    w = w_r[...]
    bb = b_r[...]
    a = a_r[0, 0]
    dp = deg_r[0][:, 0:1] + 1.0
    dn = deg_r[1][:, 0:1] + 1.0
    accp = jnp.concatenate([accl_r[0], accr_r[0]], axis=1)
    accn = jnp.concatenate([accl_r[1], accr_r[1]], axis=1)
    aggp = (accp + xv) / dp
    aggn = (accn + xv) / dn
    hp = jnp.dot(aggp, w, preferred_element_type=jnp.float32) + bb
    hn = jnp.dot(aggn, w, preferred_element_type=jnp.float32) + bb
    hp = jnp.where(hp > 0, hp, a * hp)
    hn = jnp.where(hn > 0, hn, a * hn)
    hpn = hp * lax.rsqrt(
        jnp.maximum(jnp.sum(hp * hp, axis=1, keepdims=True), 1e-30))
    hnn = hn * lax.rsqrt(
        jnp.maximum(jnp.sum(hn * hn, axis=1, keepdims=True), 1e-30))
    hcl_r[0] = hpn[:, :DH]
    hcl_r[1] = hnn[:, :DH]
    hcr_r[0] = hpn[:, DH:]
    hcr_r[1] = hnn[:, DH:]
    m = (nt_r[...] == 1.0).astype(jnp.float32)
    ph_r[...] = hp * m


def _loss_body(hcl_r, hcr_r, gl_r, gr_r, nt_r, out_r, acc_s):
    i = pl.program_id(0)
    pc = (jnp.sum(hcl_r[0] * hcl_r[1], axis=1, keepdims=True)
          + jnp.sum(hcr_r[0] * hcr_r[1], axis=1, keepdims=True))
    nl1 = (jnp.sum(hcl_r[0] * gl_r[1], axis=1, keepdims=True)
           + jnp.sum(hcr_r[0] * gr_r[1], axis=1, keepdims=True) + pc)
    nl2 = (jnp.sum(hcl_r[1] * gl_r[0], axis=1, keepdims=True)
           + jnp.sum(hcr_r[1] * gr_r[0], axis=1, keepdims=True) + pc)
    m = (nt_r[...] == 1.0).astype(jnp.float32)
    pd = jnp.sum(m * (jnp.exp(pc) + jnp.exp(nl1) + jnp.exp(nl2)))
    pp = jnp.sum(m * pc)

    @pl.when(i == 0)
    def _():
        acc_s[0] = 0.0
        acc_s[1] = 0.0

    acc_s[0] = acc_s[0] + pd
    acc_s[1] = acc_s[1] + pp

    @pl.when(i == pl.num_programs(0) - 1)
    def _():
        out_r[...] = jnp.full((1, 1), jnp.log(acc_s[0]) - acc_s[1],
                              jnp.float32)


_half_spec = pl.BlockSpec((NC, RB, DH), lambda i: (0, i, 0))

_encode = pl.pallas_call(
    _encode_body,
    grid=(N // RB,),
    in_specs=[
        pl.BlockSpec((RB, D), lambda i: (i, 0)),
        _half_spec,
        _half_spec,
        pl.BlockSpec((NC, RB, 16), lambda i: (0, i, 0)),
        pl.BlockSpec((RB, 1), lambda i: (i, 0)),
        pl.BlockSpec((D, D), lambda i: (0, 0)),
        pl.BlockSpec((1, D), lambda i: (0, 0)),
        pl.BlockSpec((1, 1), lambda i: (0, 0)),
    ],
    out_specs=[
        _half_spec,
        _half_spec,
        pl.BlockSpec((RB, D), lambda i: (i, 0)),
    ],
    out_shape=[
        jax.ShapeDtypeStruct((NC, N, DH), jnp.float32),
        jax.ShapeDtypeStruct((NC, N, DH), jnp.float32),
        jax.ShapeDtypeStruct((N, D), jnp.float32),
    ],
)

_loss = pl.pallas_call(
    _loss_body,
    grid=(N // RB,),
    in_specs=[
        _half_spec,
        _half_spec,
        _half_spec,
        _half_spec,
        pl.BlockSpec((RB, 1), lambda i: (i, 0)),
    ],
    out_specs=pl.BlockSpec((1, 1), lambda i: (0, 0)),
    out_shape=jax.ShapeDtypeStruct((1, 1), jnp.float32),
    scratch_shapes=[pltpu.SMEM((2,), jnp.float32)],
)


def kernel(x, edge_index, node_type, keep_mask, W, b, prelu_a):
    src2 = edge_index[0].reshape(NS, CPT, K)
    dst2 = edge_index[1].reshape(NS, CPT, K)
    km2 = keep_mask.reshape(NS, CPT, K)
    ntf = node_type.astype(jnp.float32).reshape(N, 1)

    (deg,) = _deg_pass()(dst2, km2)
    accl, accr = _edge_pass(N, 0)(x[:, :DH], x[:, DH:], src2, dst2, km2)
    hcl, hcr, predict_h = _encode(x, accl, accr, deg, ntf, W,
                                  b.reshape(1, D), prelu_a.reshape(1, 1))
    gl, gr = _edge_pass(2 * N, N)(hcl.reshape(2 * N, DH),
                                  hcr.reshape(2 * N, DH), src2, dst2, km2)
    loss = _loss(hcl, hcr, gl, gr, ntf)[0, 0]
    return (loss, predict_h)


# speedup vs baseline: 1.0374x; 1.0374x over previous
import functools

import jax
import jax.numpy as jnp
from jax import lax
from jax.experimental import pallas as pl
from jax.experimental.pallas import tpu as pltpu
from jax.experimental.pallas import tpu_sc as plsc

N, E, D = 10000, 320000, 128
DH = D // 2
NC, NS = 2, 16
K = 80
NB = 5
CPT = E // (K * NS)
APAD = 10112
RB = 400


@functools.lru_cache(maxsize=None)
def _edge_pass(table_rows, off):
    mesh = plsc.VectorSubcoreMesh(core_axis_name="c", subcore_axis_name="s",
                                  num_cores=NC, num_subcores=NS)
    out_type = [jax.ShapeDtypeStruct((NC, N, DH), jnp.float32)]
    scratch = (
        [pltpu.VMEM((CPT, K), jnp.int32)] * 3
        + [pltpu.VMEM((K, DH), jnp.float32)] * NB
        + [pltpu.VMEM_SHARED((APAD, DH), jnp.float32)]
        + [pltpu.SemaphoreType.DMA] * (NB + 1)
    )

    @functools.partial(
        pl.kernel, mesh=mesh, out_type=out_type, scratch_types=scratch,
        compiler_params=pltpu.CompilerParams(use_tc_tiling_on_sc=False))
    def run(*refs):
        (table, src_h, dst_h, km_h, acc_o, srcv, dstv, kmv, *rest) = refs
        rows = rest[:NB]
        acc_sh = rest[NB]
        sem_g = rest[NB + 1:2 * NB + 1]
        sem_s = rest[2 * NB + 1]
        cid = lax.axis_index("c")
        sid = lax.axis_index("s")

        pltpu.sync_copy(src_h.at[sid], srcv)
        pltpu.sync_copy(dst_h.at[sid], dstv)
        pltpu.sync_copy(km_h.at[sid], kmv)

        gsel = K // 16
        goff = jnp.full((16,), cid * off, dtype=jnp.int32)

        lanes = lax.iota(jnp.int32, 16)

        def ibody(i, _):
            c = i // gsel
            j = (i % gsel) * 16
            if off:
                srcv[c, pl.ds(j, 16)] = srcv[c, pl.ds(j, 16)] + goff
            keep = jnp.maximum(kmv[c, pl.ds(j, 16)], 1 - cid)
            dummy = (N + (i % 7) * 16) + lanes
            dstv[c, pl.ds(j, 16)] = (keep * dstv[c, pl.ds(j, 16)]
                                     + (1 - keep) * dummy)
            return 0

        lax.fori_loop(0, CPT * gsel, ibody, 0)

        z0 = sid * (APAD // NS)
        stripe = APAD // NS

        if True:
            def zbody(i, _):
                rows[0][i // (DH // 16),
                        pl.ds((i % (DH // 16)) * 16, 16)] = (
                    jnp.zeros((16,), jnp.float32))
                return 0

            lax.fori_loop(0, K * (DH // 16), zbody, 0)
            for t in range(stripe // K):
                pltpu.sync_copy(rows[0], acc_sh.at[pl.ds(z0 + t * K, K)])
            if stripe % K:
                pltpu.sync_copy(
                    rows[0].at[pl.ds(0, stripe % K)],
                    acc_sh.at[pl.ds(z0 + (stripe // K) * K, stripe % K)])
            plsc.subcore_barrier()

            for b in range(NB):
                pltpu.async_copy(table.at[srcv.at[b]], rows[b], sem_g[b])

            def mbody(p, _):
                for b in range(NB):
                    c = p * NB + b
                    pltpu.make_async_copy(table.at[srcv.at[c]], rows[b],
                                          sem_g[b]).wait()
                    pltpu.async_copy(rows[b], acc_sh.at[dstv.at[c]],
                                     sem_s, add=True).wait()
                    cn = jnp.minimum(c + NB, CPT - 1)
                    pltpu.async_copy(table.at[srcv.at[cn]], rows[b],
                                     sem_g[b])
                return 0

            lax.fori_loop(0, CPT // NB, mbody, 0)
            for b in range(NB):
                pltpu.make_async_copy(table.at[srcv.at[CPT - 1]], rows[b],
                                      sem_g[b]).wait()
            plsc.subcore_barrier()

            @pl.when(sid < NS - 1)
            def _():
                w0 = sid * 640
                pltpu.sync_copy(acc_sh.at[pl.ds(w0, 640)],
                                acc_o.at[cid, pl.ds(w0, 640)])

            @pl.when(sid == NS - 1)
            def _():
                pltpu.sync_copy(acc_sh.at[pl.ds(9600, N - 9600)],
                                acc_o.at[cid, pl.ds(9600, N - 9600)])

    return run


@functools.lru_cache(maxsize=None)
def _deg_pass():
    mesh = plsc.VectorSubcoreMesh(core_axis_name="c", subcore_axis_name="s",
                                  num_cores=NC, num_subcores=NS)
    scratch = (
        [pltpu.VMEM((CPT, K), jnp.int32)] * 2
        + [pltpu.VMEM((K, 16), jnp.float32)] * 2
        + [pltpu.VMEM_SHARED((APAD, 16), jnp.float32)]
        + [pltpu.SemaphoreType.DMA] * NB
    )

    @functools.partial(
        pl.kernel, mesh=mesh,
        out_type=[jax.ShapeDtypeStruct((NC, N, 16), jnp.float32)],
        scratch_types=scratch,
        compiler_params=pltpu.CompilerParams(use_tc_tiling_on_sc=False))
    def run(dst_h, km_h, deg_o, dstv, kmv, onesv, zdeg, deg_sh, *sem_d):
        cid = lax.axis_index("c")
        sid = lax.axis_index("s")
        pltpu.sync_copy(dst_h.at[sid], dstv)
        pltpu.sync_copy(km_h.at[sid], kmv)
        gsel = K // 16

        lanes = lax.iota(jnp.int32, 16)

        def ibody(i, _):
            c = i // gsel
            j = (i % gsel) * 16
            keep = jnp.maximum(kmv[c, pl.ds(j, 16)], 1 - cid)
            dummy = (N + (i % 7) * 16) + lanes
            dstv[c, pl.ds(j, 16)] = (keep * dstv[c, pl.ds(j, 16)]
                                     + (1 - keep) * dummy)
            return 0

        lax.fori_loop(0, CPT * gsel, ibody, 0)
        onerow = jnp.maximum(1 - lax.iota(jnp.int32, 16), 0).astype(
            jnp.float32)

        def obody(i, _):
            onesv[i, pl.ds(0, 16)] = onerow
            zdeg[i, pl.ds(0, 16)] = jnp.zeros((16,), jnp.float32)
            return 0

        lax.fori_loop(0, K, obody, 0)
        z0 = sid * (APAD // NS)
        stripe = APAD // NS
        for t in range(stripe // K):
            pltpu.sync_copy(zdeg, deg_sh.at[pl.ds(z0 + t * K, K)])
        if stripe % K:
            pltpu.sync_copy(
                zdeg.at[pl.ds(0, stripe % K)],
                deg_sh.at[pl.ds(z0 + (stripe // K) * K, stripe % K)])
        plsc.subcore_barrier()

        for b in range(NB):
            pltpu.async_copy(onesv, deg_sh.at[dstv.at[b]], sem_d[b],
                             add=True)

        def dbody(p, _):
            for b in range(NB):
                c = p * NB + b
                pltpu.make_async_copy(onesv, deg_sh.at[dstv.at[c]],
                                      sem_d[b]).wait()

                @pl.when(c + NB < CPT)
                def _():
                    pltpu.async_copy(onesv, deg_sh.at[dstv.at[c + NB]],
                                     sem_d[b], add=True)

            return 0

        lax.fori_loop(0, CPT // NB, dbody, 0)
        plsc.subcore_barrier()

        @pl.when(sid < NS - 1)
        def _():
            w0 = sid * 640
            pltpu.sync_copy(deg_sh.at[pl.ds(w0, 640)],
                            deg_o.at[cid, pl.ds(w0, 640)])

        @pl.when(sid == NS - 1)
        def _():
            pltpu.sync_copy(deg_sh.at[pl.ds(9600, N - 9600)],
                            deg_o.at[cid, pl.ds(9600, N - 9600)])

    return run


def _encode_body(x_r, accl_r, accr_r, deg_r, nt_r, w_r, b_r, a_r,
                 hcl_r, hcr_r, ph_r):
    xv = x_r[...]
    w = w_r[...]
    bb = b_r[...]
    a = a_r[0, 0]
    dp = deg_r[0][:, 0:1] + 1.0
    dn = deg_r[1][:, 0:1] + 1.0
    accp = jnp.concatenate([accl_r[0], accr_r[0]], axis=1)
    accn = jnp.concatenate([accl_r[1], accr_r[1]], axis=1)
    aggp = (accp + xv) / dp
    aggn = (accn + xv) / dn
    hp = jnp.dot(aggp, w, preferred_element_type=jnp.float32) + bb
    hn = jnp.dot(aggn, w, preferred_element_type=jnp.float32) + bb
    hp = jnp.where(hp > 0, hp, a * hp)
    hn = jnp.where(hn > 0, hn, a * hn)
    hpn = hp * lax.rsqrt(
        jnp.maximum(jnp.sum(hp * hp, axis=1, keepdims=True), 1e-30))
    hnn = hn * lax.rsqrt(
        jnp.maximum(jnp.sum(hn * hn, axis=1, keepdims=True), 1e-30))
    hcl_r[0] = hpn[:, :DH]
    hcl_r[1] = hnn[:, :DH]
    hcr_r[0] = hpn[:, DH:]
    hcr_r[1] = hnn[:, DH:]
    m = (nt_r[...] == 1.0).astype(jnp.float32)
    ph_r[...] = hp * m


def _loss_body(hcl_r, hcr_r, gl_r, gr_r, nt_r, out_r, acc_s):
    i = pl.program_id(0)
    pc = (jnp.sum(hcl_r[0] * hcl_r[1], axis=1, keepdims=True)
          + jnp.sum(hcr_r[0] * hcr_r[1], axis=1, keepdims=True))
    nl1 = (jnp.sum(hcl_r[0] * gl_r[1], axis=1, keepdims=True)
           + jnp.sum(hcr_r[0] * gr_r[1], axis=1, keepdims=True) + pc)
    nl2 = (jnp.sum(hcl_r[1] * gl_r[0], axis=1, keepdims=True)
           + jnp.sum(hcr_r[1] * gr_r[0], axis=1, keepdims=True) + pc)
    m = (nt_r[...] == 1.0).astype(jnp.float32)
    pd = jnp.sum(m * (jnp.exp(pc) + jnp.exp(nl1) + jnp.exp(nl2)))
    pp = jnp.sum(m * pc)

    @pl.when(i == 0)
    def _():
        acc_s[0] = 0.0
        acc_s[1] = 0.0

    acc_s[0] = acc_s[0] + pd
    acc_s[1] = acc_s[1] + pp

    @pl.when(i == pl.num_programs(0) - 1)
    def _():
        out_r[...] = jnp.full((1, 1), jnp.log(acc_s[0]) - acc_s[1],
                              jnp.float32)


_half_spec = pl.BlockSpec((NC, RB, DH), lambda i: (0, i, 0))

_encode = pl.pallas_call(
    _encode_body,
    grid=(N // RB,),
    in_specs=[
        pl.BlockSpec((RB, D), lambda i: (i, 0)),
        _half_spec,
        _half_spec,
        pl.BlockSpec((NC, RB, 16), lambda i: (0, i, 0)),
        pl.BlockSpec((RB, 1), lambda i: (i, 0)),
        pl.BlockSpec((D, D), lambda i: (0, 0)),
        pl.BlockSpec((1, D), lambda i: (0, 0)),
        pl.BlockSpec((1, 1), lambda i: (0, 0)),
    ],
    out_specs=[
        _half_spec,
        _half_spec,
        pl.BlockSpec((RB, D), lambda i: (i, 0)),
    ],
    out_shape=[
        jax.ShapeDtypeStruct((NC, N, DH), jnp.float32),
        jax.ShapeDtypeStruct((NC, N, DH), jnp.float32),
        jax.ShapeDtypeStruct((N, D), jnp.float32),
    ],
)

_loss = pl.pallas_call(
    _loss_body,
    grid=(N // RB,),
    in_specs=[
        _half_spec,
        _half_spec,
        _half_spec,
        _half_spec,
        pl.BlockSpec((RB, 1), lambda i: (i, 0)),
    ],
    out_specs=pl.BlockSpec((1, 1), lambda i: (0, 0)),
    out_shape=jax.ShapeDtypeStruct((1, 1), jnp.float32),
    scratch_shapes=[pltpu.SMEM((2,), jnp.float32)],
)


def kernel(x, edge_index, node_type, keep_mask, W, b, prelu_a):
    src2 = edge_index[0].reshape(NS, CPT, K)
    dst2 = edge_index[1].reshape(NS, CPT, K)
    km2 = keep_mask.reshape(NS, CPT, K)
    ntf = node_type.astype(jnp.float32).reshape(N, 1)

    (deg,) = _deg_pass()(dst2, km2)
    p1 = _edge_pass(N, 0)
    (accl,) = p1(x[:, :DH], src2, dst2, km2)
    (accr,) = p1(x[:, DH:], src2, dst2, km2)
    hcl, hcr, predict_h = _encode(x, accl, accr, deg, ntf, W,
                                  b.reshape(1, D), prelu_a.reshape(1, 1))
    p2 = _edge_pass(2 * N, N)
    (gl,) = p2(hcl.reshape(2 * N, DH), src2, dst2, km2)
    (gr,) = p2(hcr.reshape(2 * N, DH), src2, dst2, km2)
    loss = _loss(hcl, hcr, gl, gr, ntf)[0, 0]
    return (loss, predict_h)
